# baseline (speedup 1.0000x reference)
import functools

import jax
import jax.numpy as jnp
from jax import lax
from jax.experimental import pallas as pl
from jax.experimental.pallas import tpu as pltpu
from jax.experimental.pallas import tpu_sc as plsc

VOCAB = 100000
PAD = 1
L = 200
B = 1024

NC, NS, LANES = 2, 16, 16
NW = NC * NS
SENT_PER_W = B // NW
LP = 224
CHUNKS = LP // LANES
IDX_ROWS = SENT_PER_W * LP // 128


def _nb_body(toks_hbm, lcr_hbm, out_hbm, toks_v, vals_v, stamp_v, score_v, sem):
    wid = lax.axis_index("s") * NC + lax.axis_index("c")

    with jax.named_scope("stage_tokens"):
        pltpu.sync_copy(toks_hbm.at[pl.ds(wid * IDX_ROWS, IDX_ROWS)], toks_v)

    with jax.named_scope("fire_gathers"):
        def fire(j, carry):
            pltpu.async_copy(lcr_hbm.at[toks_v.at[j]], vals_v.at[j], sem)
            return carry

        lax.fori_loop(0, IDX_ROWS, fire, 0)

    with jax.named_scope("drain_gathers"):
        def drain(j, carry):
            pltpu.make_async_copy(lcr_hbm.at[toks_v.at[j]], vals_v.at[j], sem).wait()
            return carry

        lax.fori_loop(0, IDX_ROWS, drain, 0)

    lanes = lax.iota(jnp.int32, LANES)

    def sentence(s, carry):
        base = s * CHUNKS
        for k in range(CHUNKS):
            q = base + k
            row = q >> 3
            col = (q & 7) * LANES
            tok = toks_v[row, pl.ds(col, LANES)]
            marker = lanes + (s * 256 + k * LANES)
            plsc.store_scatter(stamp_v, [tok], marker)
        acc = jnp.zeros((LANES,), jnp.float32)
        for k in range(CHUNKS):
            q = base + k
            row = q >> 3
            col = (q & 7) * LANES
            tok = toks_v[row, pl.ds(col, LANES)]
            val = vals_v[row, pl.ds(col, LANES)]
            back = plsc.load_gather(stamp_v, [tok])
            marker = lanes + (s * 256 + k * LANES)
            keep = (back == marker) & (tok != PAD)
            acc = acc + jnp.where(keep, val, 0.0)
        total = jnp.sum(acc)
        plsc.store_scatter(
            score_v,
            [jnp.zeros((LANES,), jnp.int32) + s],
            jnp.broadcast_to(total, (LANES,)),
            mask=lanes == 0,
        )
        return carry

    with jax.named_scope("dedup_compute"):
        lax.fori_loop(0, SENT_PER_W, sentence, 0)
    with jax.named_scope("store_scores"):
        pltpu.sync_copy(score_v, out_hbm.at[pl.ds(wid * SENT_PER_W, SENT_PER_W)])


_nb_kernel = functools.partial(
    pl.kernel,
    out_type=jax.ShapeDtypeStruct((B,), jnp.float32),
    mesh=plsc.VectorSubcoreMesh(core_axis_name="c", subcore_axis_name="s"),
    compiler_params=pltpu.CompilerParams(needs_layout_passes=False),
    scratch_types=[
        pltpu.VMEM((IDX_ROWS, 128), jnp.int32),
        pltpu.VMEM((IDX_ROWS, 128), jnp.float32),
        pltpu.VMEM((VOCAB,), jnp.int32),
        pltpu.VMEM((SENT_PER_W,), jnp.float32),
        pltpu.SemaphoreType.DMA,
    ],
)(_nb_body)


@jax.jit
def kernel(sentences, log_count_ratio, bias):
    t = sentences.T
    t = jnp.pad(t, ((0, 0), (0, LP - L)), constant_values=PAD)
    toks = t.reshape(B * LP // 128, 128)
    scores = _nb_kernel(toks, log_count_ratio) + bias
    return jnp.stack([-scores, scores], axis=1)


# speedup vs baseline: 4.9109x; 4.9109x over previous
import functools

import jax
import jax.numpy as jnp
from jax import lax
from jax.experimental import pallas as pl
from jax.experimental.pallas import tpu as pltpu
from jax.experimental.pallas import tpu_sc as plsc

VOCAB = 100000
PAD = 1
L = 200
B = 1024

NC, NS, LANES = 2, 16, 16
NW = NC * NS
SENT_PER_W = B // NW
LP = 208
CHUNKS = LP // LANES
TOK_PER_W = SENT_PER_W * LP
HALF_TOK = TOK_PER_W // 2
HALF_SENT = SENT_PER_W // 2


def _nb_body(toks_hbm, lcr_hbm, out_hbm, toks_v, vals_v, stamp_v, score_v,
             lcr_sh, sem_t, sem_g0, sem_g1):
    cid = lax.axis_index("c")
    sid = lax.axis_index("s")
    wid = sid * NC + cid

    with jax.named_scope("stage_tokens_start"):
        tok_copy = pltpu.async_copy(
            toks_hbm.at[pl.ds(wid * TOK_PER_W, TOK_PER_W)], toks_v, sem_t)

    with jax.named_scope("stage_table"):
        @pl.when(sid == 0)
        def _():
            pltpu.sync_copy(lcr_hbm, lcr_sh)

        plsc.subcore_barrier()

    with jax.named_scope("stage_tokens_wait"):
        tok_copy.wait()

    with jax.named_scope("gather_fire"):
        g0 = pltpu.async_copy(
            lcr_sh.at[toks_v.at[pl.ds(0, HALF_TOK)]],
            vals_v.at[pl.ds(0, HALF_TOK)], sem_g0)
        g1 = pltpu.async_copy(
            lcr_sh.at[toks_v.at[pl.ds(HALF_TOK, HALF_TOK)]],
            vals_v.at[pl.ds(HALF_TOK, HALF_TOK)], sem_g1)

    lanes = lax.iota(jnp.int32, LANES)

    def sentence(s, carry):
        base = s * LP
        for k in range(CHUNKS):
            tok = toks_v[pl.ds(base + k * LANES, LANES)]
            marker = lanes + (s * 256 + k * LANES)
            plsc.store_scatter(stamp_v, [tok], marker)
        acc = jnp.zeros((LANES,), jnp.float32)
        for k in range(CHUNKS):
            tok = toks_v[pl.ds(base + k * LANES, LANES)]
            val = vals_v[pl.ds(base + k * LANES, LANES)]
            back = plsc.load_gather(stamp_v, [tok])
            marker = lanes + (s * 256 + k * LANES)
            keep = (back == marker) & (tok != PAD)
            acc = acc + jnp.where(keep, val, 0.0)
        total = jnp.sum(acc)
        plsc.store_scatter(
            score_v,
            [jnp.zeros((LANES,), jnp.int32) + s],
            jnp.broadcast_to(total, (LANES,)),
            mask=lanes == 0,
        )
        return carry

    with jax.named_scope("gather_wait0"):
        g0.wait()
    with jax.named_scope("dedup_compute0"):
        lax.fori_loop(0, HALF_SENT, sentence, 0)
    with jax.named_scope("gather_wait1"):
        g1.wait()
    with jax.named_scope("dedup_compute1"):
        lax.fori_loop(HALF_SENT, SENT_PER_W, sentence, 0)
    with jax.named_scope("store_scores"):
        pltpu.sync_copy(score_v, out_hbm.at[pl.ds(wid * SENT_PER_W, SENT_PER_W)])


_nb_kernel = functools.partial(
    pl.kernel,
    out_type=jax.ShapeDtypeStruct((B,), jnp.float32),
    mesh=plsc.VectorSubcoreMesh(core_axis_name="c", subcore_axis_name="s"),
    compiler_params=pltpu.CompilerParams(needs_layout_passes=False),
    scratch_types=[
        pltpu.VMEM((TOK_PER_W,), jnp.int32),
        pltpu.VMEM((TOK_PER_W,), jnp.float32),
        pltpu.VMEM((VOCAB,), jnp.int32),
        pltpu.VMEM((SENT_PER_W,), jnp.float32),
        pltpu.VMEM_SHARED((VOCAB,), jnp.float32),
        pltpu.SemaphoreType.DMA,
        pltpu.SemaphoreType.DMA,
        pltpu.SemaphoreType.DMA,
    ],
)(_nb_body)


@jax.jit
def kernel(sentences, log_count_ratio, bias):
    t = jnp.pad(sentences, ((0, LP - L), (0, 0)), constant_values=PAD)
    toks = t.T.reshape(B * LP)
    scores = _nb_kernel(toks, log_count_ratio) + bias
    return jnp.stack([-scores, scores], axis=1)
